# baseline (speedup 1.0000x reference)
import functools
import math

import jax
import jax.numpy as jnp
from jax import lax
from jax.experimental import pallas as pl
from jax.experimental.pallas import tpu as pltpu
from jax.experimental.pallas import tpu_sc as plsc

_H = 512
_W = 512
_N = _H * _W
_NL = 8
_FD = 4
_B = 4
_DIAG = math.sqrt(_H * _H + _W * _W)
_NS = 16
_CH = _N // _NS
_NV = _CH // 16
_BIG = 1 << 30
_LN2 = 0.6931471805599453


def _scalar(x):
    return x[0] if getattr(x, "ndim", 0) else x


def _rsumv16(x, lane, tmp):
    for sh in (8, 4, 2, 1):
        tmp[pl.ds(0, 16)] = x
        x = x + plsc.load_gather(tmp, [lane ^ sh])
    return x


def _sqrt16(x):
    i = plsc.bitcast(x, jnp.int32)
    y = plsc.bitcast(1597463007 - (i >> 1), jnp.float32)
    xh = 0.5 * x
    for _ in range(3):
        y = y * (1.5 - xh * y * y)
    return x * y


def _log1p16(t):
    x = 1.0 + t
    bits = plsc.bitcast(x, jnp.int32)
    e = (bits >> 23) - 127
    m = plsc.bitcast((bits & 8388607) | 1065353216, jnp.float32)
    big = m > 1.4142135
    m = jnp.where(big, m * 0.5, m)
    e = jnp.where(big, e + 1, e)
    s = (m - 1.0) / (m + 1.0)
    s2 = s * s
    p = s2 * (1.0 / 9.0) + (1.0 / 7.0)
    p = s2 * p + 0.2
    p = s2 * p + (1.0 / 3.0)
    p = s2 * p + 1.0
    return e.astype(jnp.float32) * _LN2 + 2.0 * s * p


_mesh = plsc.VectorSubcoreMesh(core_axis_name="c", subcore_axis_name="s")


@functools.partial(
    pl.kernel,
    out_type=(jax.ShapeDtypeStruct((_B, 16), jnp.float32),
              jax.ShapeDtypeStruct((2, _NS, 80), jnp.float32),
              jax.ShapeDtypeStruct((2, _NS, _NL, 16), jnp.int32),
              jax.ShapeDtypeStruct((2, _NS, _NL, 16), jnp.int32),
              jax.ShapeDtypeStruct((2, _NS, 16), jnp.float32)),
    mesh=_mesh,
    scratch_types=[
        pltpu.VMEM((_FD, _CH), jnp.float32),
        pltpu.VMEM((_CH,), jnp.int32),
        pltpu.VMEM((640,), jnp.float32),
        pltpu.VMEM((80,), jnp.float32),
        pltpu.VMEM((_NL, 16), jnp.int32),
        pltpu.VMEM((_NL, 16), jnp.int32),
        pltpu.VMEM((64,), jnp.float32),
        pltpu.VMEM((16,), jnp.float32),
        pltpu.VMEM((16,), jnp.float32),
        pltpu.VMEM((16,), jnp.float32),
        pltpu.VMEM((16,), jnp.int32),
        pltpu.VMEM((16,), jnp.float32),
        pltpu.VMEM((_NS, 80), jnp.float32),
        pltpu.VMEM((_NS, _NL, 16), jnp.int32),
        pltpu.VMEM((_NS, _NL, 16), jnp.int32),
        pltpu.VMEM((_NS, 16), jnp.float32),
        pltpu.SMEM((1,), jnp.int32),
    ],
    compiler_params=pltpu.CompilerParams(needs_layout_passes=False),
)
def _sc_loss(emb_hbm, inst_hbm, out_hbm, sgf, sgi1, sgi2, sgl, ev, iv,
             tblB, tbl, f1r, f2r, mu_r, cll_r, wt_r, lv, ti, lb_r,
             allf, alli1, alli2, alll, done_s):
    core = lax.axis_index("c")
    wid = lax.axis_index("s")
    rid = core * _NS + wid
    rb = pl.multiple_of(core * _NS, 16)
    base = pl.multiple_of(wid * _CH, 256)
    lane = lax.broadcasted_iota(jnp.int32, (16,), 0)
    ones = jnp.ones((16,), jnp.float32)
    zeros = jnp.zeros((16,), jnp.float32)
    bigv = jnp.full((16,), _BIG, jnp.int32)

    def one_image(bi, carry):
        b = core + 2 * bi
        pltpu.sync_copy(inst_hbm.at[b, pl.ds(base, _CH)], iv)
        for f in range(_FD):
            pltpu.sync_copy(emb_hbm.at[b, f, pl.ds(base, _CH)], ev.at[f])
        for j in range(40):
            tblB[pl.ds(j * 16, 16)] = zeros
        for l in range(_NL):
            f1r[l] = bigv
            f2r[l] = bigv
        done_s[0] = 0

        def astep(i, c):
            lab = iv[pl.ds(i * 16, 16)]
            lab16 = lab * 16 + lane
            plsc.addupdate_scatter(tblB, [lab16], ones)
            for f in range(_FD):
                e = ev[f, pl.ds(i * 16, 16)]
                plsc.addupdate_scatter(tblB, [lab16 + (128 + 128 * f)], e)

            @pl.when(done_s[0] == 0)
            def _():
                posv = base + i * 16 + lane
                for l in range(_NL):
                    x = jnp.where(lab == l, posv, _BIG)
                    f1 = f1r[l]
                    f2 = f2r[l]
                    f1r[l] = jnp.minimum(f1, x)
                    f2r[l] = jnp.minimum(f2, jnp.maximum(f1, x))

                @pl.when(i & 15 == 0)
                def _():
                    tot = zeros
                    for l in range(_NL):
                        row = tblB[pl.ds(l * 16, 16)]
                        s = _rsumv16(row, lane, lv)
                        tot = jnp.where(lane == l, s, tot)
                    needm = (tot < 2.0) & (lane < _NL)
                    nneed = _rsumv16(jnp.where(needm, 1, 0), lane, ti)
                    done_s[0] = jnp.where(nneed[0] > 0, 0, 1)
            return c

        lax.fori_loop(0, _NV, astep, 0)

        for r in range(5):
            coll = zeros
            for l in range(_NL):
                row = tblB[pl.ds(r * 128 + l * 16, 16)]
                s = _rsumv16(row, lane, lv)
                coll = jnp.where(lane == l, s, coll)
            tbl[pl.ds(r * 16, 16)] = coll

        pltpu.sync_copy(tbl, sgf.at[core, wid])
        pltpu.sync_copy(f1r, sgi1.at[core, wid])
        pltpu.sync_copy(f2r, sgi2.at[core, wid])
        plsc.subcore_barrier()
        pltpu.sync_copy(sgf.at[core], allf)
        pltpu.sync_copy(sgi1.at[core], alli1)
        pltpu.sync_copy(sgi2.at[core], alli2)

        cnt16 = zeros
        sums = [zeros] * _FD
        for t in range(_NS):
            cnt16 = cnt16 + allf[t, pl.ds(0, 16)]
            for f in range(_FD):
                sums[f] = sums[f] + allf[t, pl.ds(16 + 16 * f, 16)]

        fvec = bigv
        svec = bigv
        for l in range(_NL):
            g1 = bigv
            g2 = bigv
            for t in range(_NS):
                for src in (alli1, alli2):
                    x = src[t, l]
                    ng1 = jnp.minimum(g1, x)
                    g2 = jnp.minimum(g2, jnp.maximum(g1, x))
                    g1 = ng1
            g1s = lax.sort(g1)
            g2s = lax.sort(g2)
            m1 = g1s[0]
            m2 = jnp.minimum(g1s[1], g2s[0])
            fvec = jnp.where(lane == l, m1, fvec)
            svec = jnp.where(lane == l, m2, svec)

        presv = jnp.where(cnt16 > 0.5, 1.0, 0.0)
        ninstv = _rsumv16(presv, lane, lv)
        nnzv = jnp.maximum(
            _rsumv16(jnp.where(lane >= 1, presv, 0.0), lane, lv), 1.0)
        cntc = jnp.maximum(cnt16, 1.0)
        wtv = jnp.where(lane == 0, 0.0, presv / cntc / nnzv)
        muv = [jnp.where(lane == 0, 0.0, sums[f] / cntc) for f in range(_FD)]

        fc = jnp.where(fvec == _BIG, 0, fvec)
        sc = jnp.where(svec == _BIG, 0, svec)
        r1f = (fc >> 9).astype(jnp.float32)
        c1f = (fc & 511).astype(jnp.float32)
        r2f = (sc >> 9).astype(jnp.float32)
        c2f = (sc & 511).astype(jnp.float32)
        dii2 = (r1f - c1f) * (r1f - c1f) + (r2f - c2f) * (r2f - c2f)
        uv = r1f + c1f
        vv = r2f + c2f
        cllv = jnp.exp(_sqrt16(dii2) * (0.5 / _DIAG))

        for f in range(_FD):
            mu_r[pl.ds(f * 16, 16)] = muv[f]
        cll_r[pl.ds(0, 16)] = cllv
        wt_r[pl.ds(0, 16)] = wtv

        norm2 = zeros
        for f in range(_FD):
            norm2 = norm2 + muv[f] * muv[f]
        lregv = _log1p16(_sqrt16(norm2))
        l_regv = _rsumv16(jnp.where(presv > 0.5, lregv, 0.0), lane, lv) \
            / jnp.maximum(ninstv, 1.0) * 0.001

        acc = zeros
        accn = zeros
        for i in range(1, _NL):
            u_i = uv[i]
            v_i = vv[i]
            p_i = presv[i]
            c_i = cllv[i]
            du = uv - u_i
            dv = vv - v_i
            dij = _sqrt16(du * du + dv * dv)
            off = 1.0 - 20.0 * jnp.exp(-4.0 - (2.5 / _DIAG) * dij)
            coefrow = jnp.where(lane == i, c_i, off)
            d2m = zeros
            for f in range(_FD):
                mu_if = muv[f][i]
                dmu = muv[f] - mu_if
                d2m = d2m + dmu * dmu
            dmr = _sqrt16(d2m)
            td = jnp.maximum(3.0 - coefrow * dmr, 0.0)
            valr = _log1p16(td * td)
            mrow = (presv > 0.5) & (p_i > 0.5) & (lane != i) \
                & (lane >= 1) & (lane < _NL)
            acc = acc + jnp.where(mrow, valr, 0.0)
            accn = accn + jnp.where(mrow, 1.0, 0.0)
        npairsv = jnp.maximum(_rsumv16(accn, lane, lv), 1.0)
        l_disv = jnp.where(ninstv > 2.5,
                           _rsumv16(acc, lane, lv) / npairsv, 0.0)

        def bstep(i, lacc):
            lab = iv[pl.ds(i * 16, 16)]
            d2 = zeros
            for f in range(_FD):
                idx = lab if f == 0 else lab + (16 * f)
                mup = plsc.load_gather(mu_r, [idx])
                e = ev[f, pl.ds(i * 16, 16)]
                de = e - mup
                d2 = d2 + de * de
            cllp = plsc.load_gather(cll_r, [lab])
            wtp = plsc.load_gather(wt_r, [lab])
            t = jnp.maximum(cllp * _sqrt16(d2) - 0.5, 0.0)
            return lacc + _log1p16(t * t) * wtp

        lacc = lax.fori_loop(0, _NV, bstep, zeros)
        lb_r[pl.ds(0, 16)] = lacc
        pltpu.sync_copy(lb_r, sgl.at[core, wid])
        plsc.subcore_barrier()

        @pl.when(wid == 0)
        def _():
            pltpu.sync_copy(sgl.at[core], alll)
            tot = zeros
            for t in range(_NS):
                tot = tot + alll[t]
            l_aggv = _rsumv16(tot, lane, lv)
            lossv = l_aggv + l_disv + l_regv
            lossv = jnp.where(ninstv <= 1.5, zeros, lossv)
            lv[pl.ds(0, 16)] = lossv
            pltpu.sync_copy(lv, out_hbm.at[b])

        plsc.subcore_barrier()
        return carry

    lax.fori_loop(0, _B // 2, one_image, 0)


def kernel(emb, instance, kernel, training_mask, bboxes):
    emb3 = emb.reshape(_B, _FD, _N)
    inst2 = instance.astype(jnp.int32).reshape(_B, _N)
    out = _sc_loss(emb3, inst2)[0]
    return jnp.mean(out[:, 0])


# speedup vs baseline: 1.0494x; 1.0494x over previous
import functools
import math

import jax
import jax.numpy as jnp
from jax import lax
from jax.experimental import pallas as pl
from jax.experimental.pallas import tpu as pltpu
from jax.experimental.pallas import tpu_sc as plsc

_H = 512
_W = 512
_N = _H * _W
_NL = 8
_FD = 4
_B = 4
_DIAG = math.sqrt(_H * _H + _W * _W)
_NS = 16
_CH = _N // _NS
_NV = _CH // 16
_BIG = 1 << 30
_LN2 = 0.6931471805599453


def _scalar(x):
    return x[0] if getattr(x, "ndim", 0) else x


def _rsumv16(x, lane, tmp):
    for sh in (8, 4, 2, 1):
        tmp[pl.ds(0, 16)] = x
        x = x + plsc.load_gather(tmp, [lane ^ sh])
    return x


def _sqrt16(x):
    i = plsc.bitcast(x, jnp.int32)
    y = plsc.bitcast(1597463007 - (i >> 1), jnp.float32)
    xh = 0.5 * x
    for _ in range(2):
        y = y * (1.5 - xh * y * y)
    return x * y


def _log1p16(t):
    x = 1.0 + t
    bits = plsc.bitcast(x, jnp.int32)
    e = (bits >> 23) - 127
    m = plsc.bitcast((bits & 8388607) | 1065353216, jnp.float32)
    big = m > 1.4142135
    m = jnp.where(big, m * 0.5, m)
    e = jnp.where(big, e + 1, e)
    s = (m - 1.0) / (m + 1.0)
    s2 = s * s
    p = s2 * 0.2 + (1.0 / 3.0)
    p = s2 * p + 1.0
    return e.astype(jnp.float32) * _LN2 + 2.0 * s * p


_mesh = plsc.VectorSubcoreMesh(core_axis_name="c", subcore_axis_name="s")


@functools.partial(
    pl.kernel,
    out_type=(jax.ShapeDtypeStruct((_B, 16), jnp.float32),
              jax.ShapeDtypeStruct((2, _NS, 80), jnp.float32),
              jax.ShapeDtypeStruct((2, _NS, _NL, 16), jnp.int32),
              jax.ShapeDtypeStruct((2, _NS, _NL, 16), jnp.int32),
              jax.ShapeDtypeStruct((2, _NS, 16), jnp.float32)),
    mesh=_mesh,
    scratch_types=[
        pltpu.VMEM((_FD, _CH), jnp.float32),
        pltpu.VMEM((_CH,), jnp.int32),
        pltpu.VMEM((640,), jnp.float32),
        pltpu.VMEM((80,), jnp.float32),
        pltpu.VMEM((_NL, 16), jnp.int32),
        pltpu.VMEM((_NL, 16), jnp.int32),
        pltpu.VMEM((64,), jnp.float32),
        pltpu.VMEM((16,), jnp.float32),
        pltpu.VMEM((16,), jnp.float32),
        pltpu.VMEM((16,), jnp.float32),
        pltpu.VMEM((16,), jnp.int32),
        pltpu.VMEM((16,), jnp.float32),
        pltpu.VMEM((_NS, 80), jnp.float32),
        pltpu.VMEM((_NS, _NL, 16), jnp.int32),
        pltpu.VMEM((_NS, _NL, 16), jnp.int32),
        pltpu.VMEM((_NS, 16), jnp.float32),
        pltpu.SMEM((1,), jnp.int32),
        pltpu.SemaphoreType.DMA,
    ],
    compiler_params=pltpu.CompilerParams(needs_layout_passes=False),
)
def _sc_loss(emb_hbm, inst_hbm, out_hbm, sgf, sgi1, sgi2, sgl, ev, iv,
             tblB, tbl, f1r, f2r, mu_r, cll_r, wt_r, lv, ti, lb_r,
             allf, alli1, alli2, alll, done_s, dsem):
    core = lax.axis_index("c")
    wid = lax.axis_index("s")
    rid = core * _NS + wid
    rb = pl.multiple_of(core * _NS, 16)
    base = pl.multiple_of(wid * _CH, 256)
    lane = lax.broadcasted_iota(jnp.int32, (16,), 0)
    ones = jnp.ones((16,), jnp.float32)
    zeros = jnp.zeros((16,), jnp.float32)
    bigv = jnp.full((16,), _BIG, jnp.int32)

    def one_image(bi, carry):
        b = core + 2 * bi
        descs = [pltpu.async_copy(inst_hbm.at[b, pl.ds(base, _CH)], iv, dsem)]
        for f in range(_FD):
            descs.append(
                pltpu.async_copy(emb_hbm.at[b, f, pl.ds(base, _CH)],
                                 ev.at[f], dsem))
        for dsc in descs:
            dsc.wait()
        for j in range(40):
            tblB[pl.ds(j * 16, 16)] = zeros
        for l in range(_NL):
            f1r[l] = bigv
            f2r[l] = bigv
        done_s[0] = 0

        def astep(i, c):
            lab = iv[pl.ds(i * 16, 16)]
            lab16 = lab * 16 + lane
            plsc.addupdate_scatter(tblB, [lab16], ones)
            for f in range(_FD):
                e = ev[f, pl.ds(i * 16, 16)]
                plsc.addupdate_scatter(tblB, [lab16 + (128 + 128 * f)], e)

            @pl.when(done_s[0] == 0)
            def _():
                posv = base + i * 16 + lane
                for l in range(_NL):
                    x = jnp.where(lab == l, posv, _BIG)
                    f1 = f1r[l]
                    f2 = f2r[l]
                    f1r[l] = jnp.minimum(f1, x)
                    f2r[l] = jnp.minimum(f2, jnp.maximum(f1, x))

                @pl.when(i & 15 == 0)
                def _():
                    tot = zeros
                    for l in range(_NL):
                        row = tblB[pl.ds(l * 16, 16)]
                        s = _rsumv16(row, lane, lv)
                        tot = jnp.where(lane == l, s, tot)
                    needm = (tot < 2.0) & (lane < _NL)
                    nneed = _rsumv16(jnp.where(needm, 1, 0), lane, ti)
                    done_s[0] = jnp.where(nneed[0] > 0, 0, 1)
            return c

        lax.fori_loop(0, _NV, astep, 0)

        for r in range(5):
            coll = zeros
            for l in range(_NL):
                row = tblB[pl.ds(r * 128 + l * 16, 16)]
                s = _rsumv16(row, lane, lv)
                coll = jnp.where(lane == l, s, coll)
            tbl[pl.ds(r * 16, 16)] = coll

        pltpu.sync_copy(tbl, sgf.at[core, wid])
        pltpu.sync_copy(f1r, sgi1.at[core, wid])
        pltpu.sync_copy(f2r, sgi2.at[core, wid])
        plsc.subcore_barrier()
        pltpu.sync_copy(sgf.at[core], allf)
        pltpu.sync_copy(sgi1.at[core], alli1)
        pltpu.sync_copy(sgi2.at[core], alli2)

        cnt16 = zeros
        sums = [zeros] * _FD
        for t in range(_NS):
            cnt16 = cnt16 + allf[t, pl.ds(0, 16)]
            for f in range(_FD):
                sums[f] = sums[f] + allf[t, pl.ds(16 + 16 * f, 16)]

        fvec = bigv
        svec = bigv
        for l in range(_NL):
            g1 = bigv
            g2 = bigv
            for t in range(_NS):
                for src in (alli1, alli2):
                    x = src[t, l]
                    ng1 = jnp.minimum(g1, x)
                    g2 = jnp.minimum(g2, jnp.maximum(g1, x))
                    g1 = ng1
            g1s = lax.sort(g1)
            g2s = lax.sort(g2)
            m1 = g1s[0]
            m2 = jnp.minimum(g1s[1], g2s[0])
            fvec = jnp.where(lane == l, m1, fvec)
            svec = jnp.where(lane == l, m2, svec)

        presv = jnp.where(cnt16 > 0.5, 1.0, 0.0)
        ninstv = _rsumv16(presv, lane, lv)
        nnzv = jnp.maximum(
            _rsumv16(jnp.where(lane >= 1, presv, 0.0), lane, lv), 1.0)
        cntc = jnp.maximum(cnt16, 1.0)
        wtv = jnp.where(lane == 0, 0.0, presv / cntc / nnzv)
        muv = [jnp.where(lane == 0, 0.0, sums[f] / cntc) for f in range(_FD)]

        fc = jnp.where(fvec == _BIG, 0, fvec)
        sc = jnp.where(svec == _BIG, 0, svec)
        r1f = (fc >> 9).astype(jnp.float32)
        c1f = (fc & 511).astype(jnp.float32)
        r2f = (sc >> 9).astype(jnp.float32)
        c2f = (sc & 511).astype(jnp.float32)
        dii2 = (r1f - c1f) * (r1f - c1f) + (r2f - c2f) * (r2f - c2f)
        uv = r1f + c1f
        vv = r2f + c2f
        cllv = jnp.exp(_sqrt16(dii2) * (0.5 / _DIAG))

        for f in range(_FD):
            mu_r[pl.ds(f * 16, 16)] = muv[f]
        cll_r[pl.ds(0, 16)] = cllv
        wt_r[pl.ds(0, 16)] = wtv

        norm2 = zeros
        for f in range(_FD):
            norm2 = norm2 + muv[f] * muv[f]
        lregv = _log1p16(_sqrt16(norm2))
        l_regv = _rsumv16(jnp.where(presv > 0.5, lregv, 0.0), lane, lv) \
            / jnp.maximum(ninstv, 1.0) * 0.001

        acc = zeros
        accn = zeros
        for i in range(1, _NL):
            u_i = uv[i]
            v_i = vv[i]
            p_i = presv[i]
            c_i = cllv[i]
            du = uv - u_i
            dv = vv - v_i
            dij = _sqrt16(du * du + dv * dv)
            off = 1.0 - 20.0 * jnp.exp(-4.0 - (2.5 / _DIAG) * dij)
            coefrow = jnp.where(lane == i, c_i, off)
            d2m = zeros
            for f in range(_FD):
                mu_if = muv[f][i]
                dmu = muv[f] - mu_if
                d2m = d2m + dmu * dmu
            dmr = _sqrt16(d2m)
            td = jnp.maximum(3.0 - coefrow * dmr, 0.0)
            valr = _log1p16(td * td)
            mrow = (presv > 0.5) & (p_i > 0.5) & (lane != i) \
                & (lane >= 1) & (lane < _NL)
            acc = acc + jnp.where(mrow, valr, 0.0)
            accn = accn + jnp.where(mrow, 1.0, 0.0)
        npairsv = jnp.maximum(_rsumv16(accn, lane, lv), 1.0)
        l_disv = jnp.where(ninstv > 2.5,
                           _rsumv16(acc, lane, lv) / npairsv, 0.0)

        def bstep(i, lacc):
            for u in range(2):
                o = i * 32 + u * 16
                lab = iv[pl.ds(o, 16)]
                d2 = zeros
                for f in range(_FD):
                    idx = lab if f == 0 else lab + (16 * f)
                    mup = plsc.load_gather(mu_r, [idx])
                    e = ev[f, pl.ds(o, 16)]
                    de = e - mup
                    d2 = d2 + de * de
                cllp = plsc.load_gather(cll_r, [lab])
                wtp = plsc.load_gather(wt_r, [lab])
                t = jnp.maximum(cllp * _sqrt16(d2) - 0.5, 0.0)
                lacc = lacc + _log1p16(t * t) * wtp
            return lacc

        lacc = lax.fori_loop(0, _NV // 2, bstep, zeros)
        lb_r[pl.ds(0, 16)] = lacc
        pltpu.sync_copy(lb_r, sgl.at[core, wid])
        plsc.subcore_barrier()

        @pl.when(wid == 0)
        def _():
            pltpu.sync_copy(sgl.at[core], alll)
            tot = zeros
            for t in range(_NS):
                tot = tot + alll[t]
            l_aggv = _rsumv16(tot, lane, lv)
            lossv = l_aggv + l_disv + l_regv
            lossv = jnp.where(ninstv <= 1.5, zeros, lossv)
            lv[pl.ds(0, 16)] = lossv
            pltpu.sync_copy(lv, out_hbm.at[b])

        plsc.subcore_barrier()
        return carry

    lax.fori_loop(0, _B // 2, one_image, 0)


def kernel(emb, instance, kernel, training_mask, bboxes):
    emb3 = emb.reshape(_B, _FD, _N)
    inst2 = instance.astype(jnp.int32).reshape(_B, _N)
    out = _sc_loss(emb3, inst2)[0]
    return jnp.mean(out[:, 0])


# speedup vs baseline: 1.0515x; 1.0020x over previous
import functools
import math

import jax
import jax.numpy as jnp
from jax import lax
from jax.experimental import pallas as pl
from jax.experimental.pallas import tpu as pltpu
from jax.experimental.pallas import tpu_sc as plsc

_H = 512
_W = 512
_N = _H * _W
_NL = 8
_FD = 4
_B = 4
_DIAG = math.sqrt(_H * _H + _W * _W)
_NS = 16
_CH = _N // _NS
_NV = _CH // 16
_BIG = 1 << 30
_LN2 = 0.6931471805599453


def _scalar(x):
    return x[0] if getattr(x, "ndim", 0) else x


def _rsumv16(x, lane, tmp):
    for sh in (8, 4, 2, 1):
        tmp[pl.ds(0, 16)] = x
        x = x + plsc.load_gather(tmp, [lane ^ sh])
    return x


def _sqrt16(x):
    i = plsc.bitcast(x, jnp.int32)
    y = plsc.bitcast(1597463007 - (i >> 1), jnp.float32)
    xh = 0.5 * x
    for _ in range(2):
        y = y * (1.5 - xh * y * y)
    return x * y


def _log1p16(t):
    x = 1.0 + t
    bits = plsc.bitcast(x, jnp.int32)
    e = (bits >> 23) - 127
    m = plsc.bitcast((bits & 8388607) | 1065353216, jnp.float32)
    big = m > 1.4142135
    m = jnp.where(big, m * 0.5, m)
    e = jnp.where(big, e + 1, e)
    s = (m - 1.0) / (m + 1.0)
    s2 = s * s
    p = s2 * 0.2 + (1.0 / 3.0)
    p = s2 * p + 1.0
    return e.astype(jnp.float32) * _LN2 + 2.0 * s * p


_mesh = plsc.VectorSubcoreMesh(core_axis_name="c", subcore_axis_name="s")


@functools.partial(
    pl.kernel,
    out_type=(jax.ShapeDtypeStruct((_B, 16), jnp.float32),
              jax.ShapeDtypeStruct((2, _NS, 80), jnp.float32),
              jax.ShapeDtypeStruct((2, _NS, _NL, 16), jnp.int32),
              jax.ShapeDtypeStruct((2, _NS, _NL, 16), jnp.int32),
              jax.ShapeDtypeStruct((2, _NS, 16), jnp.float32)),
    mesh=_mesh,
    scratch_types=[
        pltpu.VMEM((_FD, _CH), jnp.float32),
        pltpu.VMEM((_CH,), jnp.int32),
        pltpu.VMEM((640,), jnp.float32),
        pltpu.VMEM((80,), jnp.float32),
        pltpu.VMEM((_NL, 16), jnp.int32),
        pltpu.VMEM((_NL, 16), jnp.int32),
        pltpu.VMEM((64,), jnp.float32),
        pltpu.VMEM((16,), jnp.float32),
        pltpu.VMEM((16,), jnp.float32),
        pltpu.VMEM((16,), jnp.float32),
        pltpu.VMEM((16,), jnp.int32),
        pltpu.VMEM((16,), jnp.float32),
        pltpu.VMEM((_NS, 80), jnp.float32),
        pltpu.VMEM((_NS, _NL, 16), jnp.int32),
        pltpu.VMEM((_NS, _NL, 16), jnp.int32),
        pltpu.VMEM((_NS, 16), jnp.float32),
        pltpu.SMEM((1,), jnp.int32),
        pltpu.SemaphoreType.DMA,
    ],
    compiler_params=pltpu.CompilerParams(needs_layout_passes=False),
)
def _sc_loss(emb_hbm, inst_hbm, out_hbm, sgf, sgi1, sgi2, sgl, ev, iv,
             tblB, tbl, f1r, f2r, mu_r, cll_r, wt_r, lv, ti, lb_r,
             allf, alli1, alli2, alll, done_s, dsem):
    core = lax.axis_index("c")
    wid = lax.axis_index("s")
    rid = core * _NS + wid
    rb = pl.multiple_of(core * _NS, 16)
    base = pl.multiple_of(wid * _CH, 256)
    lane = lax.broadcasted_iota(jnp.int32, (16,), 0)
    ones = jnp.ones((16,), jnp.float32)
    zeros = jnp.zeros((16,), jnp.float32)
    bigv = jnp.full((16,), _BIG, jnp.int32)

    def one_image(bi, carry):
        b = core + 2 * bi
        descs = [pltpu.async_copy(inst_hbm.at[b, pl.ds(base, _CH)], iv, dsem)]
        for f in range(_FD):
            descs.append(
                pltpu.async_copy(emb_hbm.at[b, f, pl.ds(base, _CH)],
                                 ev.at[f], dsem))
        for dsc in descs:
            dsc.wait()
        for j in range(40):
            tblB[pl.ds(j * 16, 16)] = zeros
        for l in range(_NL):
            f1r[l] = bigv
            f2r[l] = bigv
        done_s[0] = 0

        def astep(i, c):
            lab = iv[pl.ds(i * 16, 16)]
            lab16 = lab * 16 + lane
            plsc.addupdate_scatter(tblB, [lab16], ones)
            for f in range(_FD):
                e = ev[f, pl.ds(i * 16, 16)]
                plsc.addupdate_scatter(tblB, [lab16 + (128 + 128 * f)], e)

            @pl.when(done_s[0] == 0)
            def _():
                posv = base + i * 16 + lane
                for l in range(_NL):
                    x = jnp.where(lab == l, posv, _BIG)
                    f1 = f1r[l]
                    f2 = f2r[l]
                    f1r[l] = jnp.minimum(f1, x)
                    f2r[l] = jnp.minimum(f2, jnp.maximum(f1, x))

                @pl.when(i & 15 == 0)
                def _():
                    tot = zeros
                    for l in range(_NL):
                        row = tblB[pl.ds(l * 16, 16)]
                        s = _rsumv16(row, lane, lv)
                        tot = jnp.where(lane == l, s, tot)
                    needm = (tot < 2.0) & (lane < _NL)
                    nneed = _rsumv16(jnp.where(needm, 1, 0), lane, ti)
                    done_s[0] = jnp.where(nneed[0] > 0, 0, 1)
            return c

        lax.fori_loop(0, _NV, astep, 0)

        for r in range(5):
            coll = zeros
            for l in range(_NL):
                row = tblB[pl.ds(r * 128 + l * 16, 16)]
                s = _rsumv16(row, lane, lv)
                coll = jnp.where(lane == l, s, coll)
            tbl[pl.ds(r * 16, 16)] = coll

        pltpu.sync_copy(tbl, sgf.at[core, wid])
        pltpu.sync_copy(f1r, sgi1.at[core, wid])
        pltpu.sync_copy(f2r, sgi2.at[core, wid])
        plsc.subcore_barrier()
        pltpu.sync_copy(sgf.at[core], allf)
        pltpu.sync_copy(sgi1.at[core], alli1)
        pltpu.sync_copy(sgi2.at[core], alli2)

        cnt16 = zeros
        sums = [zeros] * _FD
        for t in range(_NS):
            cnt16 = cnt16 + allf[t, pl.ds(0, 16)]
            for f in range(_FD):
                sums[f] = sums[f] + allf[t, pl.ds(16 + 16 * f, 16)]

        fvec = bigv
        svec = bigv
        for l in range(_NL):
            g1 = bigv
            g2 = bigv
            for t in range(_NS):
                for src in (alli1, alli2):
                    x = src[t, l]
                    ng1 = jnp.minimum(g1, x)
                    g2 = jnp.minimum(g2, jnp.maximum(g1, x))
                    g1 = ng1
            g1s = lax.sort(g1)
            g2s = lax.sort(g2)
            m1 = g1s[0]
            m2 = jnp.minimum(g1s[1], g2s[0])
            fvec = jnp.where(lane == l, m1, fvec)
            svec = jnp.where(lane == l, m2, svec)

        presv = jnp.where(cnt16 > 0.5, 1.0, 0.0)
        ninstv = _rsumv16(presv, lane, lv)
        nnzv = jnp.maximum(
            _rsumv16(jnp.where(lane >= 1, presv, 0.0), lane, lv), 1.0)
        cntc = jnp.maximum(cnt16, 1.0)
        wtv = jnp.where(lane == 0, 0.0, presv / cntc / nnzv)
        muv = [jnp.where(lane == 0, 0.0, sums[f] / cntc) for f in range(_FD)]

        fc = jnp.where(fvec == _BIG, 0, fvec)
        sc = jnp.where(svec == _BIG, 0, svec)
        r1f = (fc >> 9).astype(jnp.float32)
        c1f = (fc & 511).astype(jnp.float32)
        r2f = (sc >> 9).astype(jnp.float32)
        c2f = (sc & 511).astype(jnp.float32)
        dii2 = (r1f - c1f) * (r1f - c1f) + (r2f - c2f) * (r2f - c2f)
        uv = r1f + c1f
        vv = r2f + c2f
        cllv = jnp.exp(_sqrt16(dii2) * (0.5 / _DIAG))

        for f in range(_FD):
            mu_r[pl.ds(f * 16, 16)] = muv[f]
        cll_r[pl.ds(0, 16)] = cllv
        wt_r[pl.ds(0, 16)] = wtv

        norm2 = zeros
        for f in range(_FD):
            norm2 = norm2 + muv[f] * muv[f]
        lregv = _log1p16(_sqrt16(norm2))
        l_regv = _rsumv16(jnp.where(presv > 0.5, lregv, 0.0), lane, lv) \
            / jnp.maximum(ninstv, 1.0) * 0.001

        acc = zeros
        accn = zeros
        for i in range(1, _NL):
            u_i = uv[i]
            v_i = vv[i]
            p_i = presv[i]
            c_i = cllv[i]
            du = uv - u_i
            dv = vv - v_i
            dij = _sqrt16(du * du + dv * dv)
            off = 1.0 - 20.0 * jnp.exp(-4.0 - (2.5 / _DIAG) * dij)
            coefrow = jnp.where(lane == i, c_i, off)
            d2m = zeros
            for f in range(_FD):
                mu_if = muv[f][i]
                dmu = muv[f] - mu_if
                d2m = d2m + dmu * dmu
            dmr = _sqrt16(d2m)
            td = jnp.maximum(3.0 - coefrow * dmr, 0.0)
            valr = _log1p16(td * td)
            mrow = (presv > 0.5) & (p_i > 0.5) & (lane != i) \
                & (lane >= 1) & (lane < _NL)
            acc = acc + jnp.where(mrow, valr, 0.0)
            accn = accn + jnp.where(mrow, 1.0, 0.0)
        npairsv = jnp.maximum(_rsumv16(accn, lane, lv), 1.0)
        l_disv = jnp.where(ninstv > 2.5,
                           _rsumv16(acc, lane, lv) / npairsv, 0.0)

        def bstep(i, laccs):
            outs = []
            for u in range(4):
                o = i * 64 + u * 16
                lab = iv[pl.ds(o, 16)]
                d2 = zeros
                for f in range(_FD):
                    idx = lab if f == 0 else lab + (16 * f)
                    mup = plsc.load_gather(mu_r, [idx])
                    e = ev[f, pl.ds(o, 16)]
                    de = e - mup
                    d2 = d2 + de * de
                cllp = plsc.load_gather(cll_r, [lab])
                wtp = plsc.load_gather(wt_r, [lab])
                t = jnp.maximum(cllp * _sqrt16(d2) - 0.5, 0.0)
                outs.append(laccs[u] + _log1p16(t * t) * wtp)
            return tuple(outs)

        laccs = lax.fori_loop(0, _NV // 4, bstep,
                              (zeros, zeros, zeros, zeros))
        lacc = (laccs[0] + laccs[1]) + (laccs[2] + laccs[3])
        lb_r[pl.ds(0, 16)] = lacc
        pltpu.sync_copy(lb_r, sgl.at[core, wid])
        plsc.subcore_barrier()

        @pl.when(wid == 0)
        def _():
            pltpu.sync_copy(sgl.at[core], alll)
            tot = zeros
            for t in range(_NS):
                tot = tot + alll[t]
            l_aggv = _rsumv16(tot, lane, lv)
            lossv = l_aggv + l_disv + l_regv
            lossv = jnp.where(ninstv <= 1.5, zeros, lossv)
            lv[pl.ds(0, 16)] = lossv
            pltpu.sync_copy(lv, out_hbm.at[b])

        plsc.subcore_barrier()
        return carry

    lax.fori_loop(0, _B // 2, one_image, 0)


def kernel(emb, instance, kernel, training_mask, bboxes):
    emb3 = emb.reshape(_B, _FD, _N)
    inst2 = instance.astype(jnp.int32).reshape(_B, _N)
    out = _sc_loss(emb3, inst2)[0]
    return jnp.mean(out[:, 0])
